# baseline (speedup 1.0000x reference)
import functools

import jax
import jax.numpy as jnp
from jax import lax
from jax.experimental import pallas as pl
from jax.experimental.pallas import tpu as pltpu
from jax.experimental.pallas import tpu_sc as plsc

EXPERT_NUM = 1000000
EXPERT_DIM = 64
PAIR_DIM = 2 * EXPERT_DIM
BATCH = 16384

NC = 2
NS = 16
NW = NC * NS
CHUNK = 128
ROWS_PER_W = BATCH // NW
N_CHUNK = ROWS_PER_W // CHUNK


def _gather_body(pairs_hbm, idx_hbm, out_hbm, idx_v, rows_v, sem):
    wid = lax.axis_index("s") * NC + lax.axis_index("c")
    blk = wid * N_CHUNK
    pltpu.sync_copy(idx_hbm.at[pl.ds(blk, N_CHUNK)], idx_v)
    copies = [
        pltpu.async_copy(
            pairs_hbm.at[idx_v.at[j]],
            rows_v.at[pl.ds(j * CHUNK, CHUNK)],
            sem,
        )
        for j in range(N_CHUNK)
    ]
    for c in copies:
        c.wait()
    pltpu.sync_copy(rows_v, out_hbm.at[pl.ds(wid * ROWS_PER_W, ROWS_PER_W)])


@functools.cache
def _sc_gather_fn():
    return pl.kernel(
        _gather_body,
        out_type=jax.ShapeDtypeStruct((BATCH, PAIR_DIM), jnp.float32),
        mesh=plsc.VectorSubcoreMesh(
            core_axis_name="c", subcore_axis_name="s", num_cores=NC, num_subcores=NS
        ),
        scratch_types=[
            pltpu.VMEM((N_CHUNK, CHUNK), jnp.int32),
            pltpu.VMEM((ROWS_PER_W, PAIR_DIM), jnp.float32),
            pltpu.SemaphoreType.DMA,
        ],
    )


def _linear_body(pair_ref, ids_ref, w_ref, b_ref, o_ref):
    odd = (ids_ref[...] & 1) == 1
    x = jnp.where(odd, pair_ref[:, EXPERT_DIM:], pair_ref[:, :EXPERT_DIM])
    o_ref[...] = (
        lax.dot_general(
            x,
            w_ref[...],
            (((1,), (1,)), ((), ())),
            preferred_element_type=jnp.float32,
        )
        + b_ref[...]
    )


_BLK = 2048


def _tc_linear(pairs, ids2d, W, b2d):
    return pl.pallas_call(
        _linear_body,
        grid=(BATCH // _BLK,),
        in_specs=[
            pl.BlockSpec((_BLK, PAIR_DIM), lambda i: (i, 0)),
            pl.BlockSpec((_BLK, 1), lambda i: (i, 0)),
            pl.BlockSpec((EXPERT_DIM, EXPERT_DIM), lambda i: (0, 0)),
            pl.BlockSpec((1, EXPERT_DIM), lambda i: (0, 0)),
        ],
        out_specs=pl.BlockSpec((_BLK, EXPERT_DIM), lambda i: (i, 0)),
        out_shape=jax.ShapeDtypeStruct((BATCH, EXPERT_DIM), jnp.float32),
    )(pairs, ids2d, W, b2d)


@jax.jit
def kernel(expert_id, table, W, b):
    ids = expert_id.astype(jnp.int32)
    pair_table = table.reshape(EXPERT_NUM // 2, PAIR_DIM)
    idx = (ids >> 1).reshape(BATCH // CHUNK, CHUNK)
    xpair = _sc_gather_fn()(pair_table, idx)
    return _tc_linear(xpair, ids.reshape(BATCH, 1), W, b.reshape(1, EXPERT_DIM))


# speedup vs baseline: 2.1257x; 2.1257x over previous
import functools

import jax
import jax.numpy as jnp
from jax import lax
from jax.experimental import pallas as pl
from jax.experimental.pallas import tpu as pltpu
from jax.experimental.pallas import tpu_sc as plsc

EXPERT_NUM = 1000000
EXPERT_DIM = 64
PAD_DIM = 128
BATCH = 16384

NC = 2
NS = 16
NW = NC * NS
CHUNK = 128
ROWS_PER_W = BATCH // NW
N_CHUNK = ROWS_PER_W // CHUNK

_TBL = 8192


def _tp_body(in_ref, o_ref):
    o_ref[:, :EXPERT_DIM] = jnp.swapaxes(in_ref[...], 0, 1)


def _tc_transpose(tableT):
    grid = (EXPERT_NUM + _TBL - 1) // _TBL
    return pl.pallas_call(
        _tp_body,
        grid=(grid,),
        in_specs=[pl.BlockSpec((EXPERT_DIM, _TBL), lambda i: (0, i))],
        out_specs=pl.BlockSpec((_TBL, PAD_DIM), lambda i: (i, 0)),
        out_shape=jax.ShapeDtypeStruct((EXPERT_NUM, PAD_DIM), jnp.float32),
    )(tableT)


def _gather_body(padded_hbm, idx_hbm, out_hbm, idx_v, rows_v, sem):
    wid = lax.axis_index("s") * NC + lax.axis_index("c")
    blk = wid * N_CHUNK
    pltpu.sync_copy(idx_hbm.at[pl.ds(blk, N_CHUNK)], idx_v)
    copies = [
        pltpu.async_copy(
            padded_hbm.at[idx_v.at[j]],
            rows_v.at[pl.ds(j * CHUNK, CHUNK)],
            sem,
        )
        for j in range(N_CHUNK)
    ]
    for c in copies:
        c.wait()
    pltpu.sync_copy(rows_v, out_hbm.at[pl.ds(wid * ROWS_PER_W, ROWS_PER_W)])


@functools.cache
def _sc_gather_fn():
    return pl.kernel(
        _gather_body,
        out_type=jax.ShapeDtypeStruct((BATCH, PAD_DIM), jnp.float32),
        mesh=plsc.VectorSubcoreMesh(
            core_axis_name="c", subcore_axis_name="s", num_cores=NC, num_subcores=NS
        ),
        scratch_types=[
            pltpu.VMEM((N_CHUNK, CHUNK), jnp.int32),
            pltpu.VMEM((ROWS_PER_W, PAD_DIM), jnp.float32),
            pltpu.SemaphoreType.DMA,
        ],
    )


def _linear_body(pad_ref, w_ref, b_ref, o_ref):
    o_ref[...] = (
        lax.dot_general(
            pad_ref[:, :EXPERT_DIM],
            w_ref[...],
            (((1,), (1,)), ((), ())),
            preferred_element_type=jnp.float32,
        )
        + b_ref[...]
    )


_BLK = 2048


def _tc_linear(xpad, W, b2d):
    return pl.pallas_call(
        _linear_body,
        grid=(BATCH // _BLK,),
        in_specs=[
            pl.BlockSpec((_BLK, PAD_DIM), lambda i: (i, 0)),
            pl.BlockSpec((EXPERT_DIM, EXPERT_DIM), lambda i: (0, 0)),
            pl.BlockSpec((1, EXPERT_DIM), lambda i: (0, 0)),
        ],
        out_specs=pl.BlockSpec((_BLK, EXPERT_DIM), lambda i: (i, 0)),
        out_shape=jax.ShapeDtypeStruct((BATCH, EXPERT_DIM), jnp.float32),
    )(xpad, W, b2d)


@jax.jit
def kernel(expert_id, table, W, b):
    ids = expert_id.astype(jnp.int32)
    padded = _tc_transpose(table.T)
    idx = ids.reshape(BATCH // CHUNK, CHUNK)
    xpad = _sc_gather_fn()(padded, idx)
    return _tc_linear(xpad, W, b.reshape(1, EXPERT_DIM))


# speedup vs baseline: 2.1307x; 1.0023x over previous
import functools

import jax
import jax.numpy as jnp
from jax import lax
from jax.experimental import pallas as pl
from jax.experimental.pallas import tpu as pltpu
from jax.experimental.pallas import tpu_sc as plsc

EXPERT_NUM = 1000000
EXPERT_DIM = 64
PAIR_DIM = 2 * EXPERT_DIM
N_PAIRS = EXPERT_NUM // 2
BATCH = 16384

NC = 2
NS = 16
NW = NC * NS
CHUNK = 128
ROWS_PER_W = BATCH // NW
N_CHUNK = ROWS_PER_W // CHUNK

_TBL = 8192


_TGRID = (EXPERT_NUM + _TBL - 1) // _TBL
_HALF = _TBL // 2
N_PROWS = _TGRID * _HALF


def _tp_body(in_ref, o_ref):
    y = jnp.swapaxes(in_ref[...], 0, 1)
    o_ref[:, :EXPERT_DIM] = y[:_HALF]
    o_ref[:, EXPERT_DIM:] = y[_HALF:]


def _tc_transpose(tableT):
    return pl.pallas_call(
        _tp_body,
        grid=(_TGRID,),
        in_specs=[pl.BlockSpec((EXPERT_DIM, _TBL), lambda i: (0, i))],
        out_specs=pl.BlockSpec((_HALF, PAIR_DIM), lambda i: (i, 0)),
        out_shape=jax.ShapeDtypeStruct((N_PROWS, PAIR_DIM), jnp.float32),
    )(tableT)


def _gather_body(pairs_hbm, idx_hbm, out_hbm, idx_v, rows_v, sem):
    wid = lax.axis_index("s") * NC + lax.axis_index("c")
    blk = wid * N_CHUNK
    pltpu.sync_copy(idx_hbm.at[pl.ds(blk, N_CHUNK)], idx_v)
    copies = [
        pltpu.async_copy(
            pairs_hbm.at[idx_v.at[j]],
            rows_v.at[pl.ds(j * CHUNK, CHUNK)],
            sem,
        )
        for j in range(N_CHUNK)
    ]
    for c in copies:
        c.wait()
    pltpu.sync_copy(rows_v, out_hbm.at[pl.ds(wid * ROWS_PER_W, ROWS_PER_W)])


@functools.cache
def _sc_gather_fn():
    return pl.kernel(
        _gather_body,
        out_type=jax.ShapeDtypeStruct((BATCH, PAIR_DIM), jnp.float32),
        mesh=plsc.VectorSubcoreMesh(
            core_axis_name="c", subcore_axis_name="s", num_cores=NC, num_subcores=NS
        ),
        scratch_types=[
            pltpu.VMEM((N_CHUNK, CHUNK), jnp.int32),
            pltpu.VMEM((ROWS_PER_W, PAIR_DIM), jnp.float32),
            pltpu.SemaphoreType.DMA,
        ],
    )


def _linear_body(pair_ref, half_ref, w_ref, b_ref, o_ref):
    odd = half_ref[...] == 1
    x = jnp.where(odd, pair_ref[:, EXPERT_DIM:], pair_ref[:, :EXPERT_DIM])
    o_ref[...] = (
        lax.dot_general(
            x,
            w_ref[...],
            (((1,), (1,)), ((), ())),
            preferred_element_type=jnp.float32,
        )
        + b_ref[...]
    )


_BLK = 2048


def _tc_linear(pairs, half2d, W, b2d):
    return pl.pallas_call(
        _linear_body,
        grid=(BATCH // _BLK,),
        in_specs=[
            pl.BlockSpec((_BLK, PAIR_DIM), lambda i: (i, 0)),
            pl.BlockSpec((_BLK, 1), lambda i: (i, 0)),
            pl.BlockSpec((EXPERT_DIM, EXPERT_DIM), lambda i: (0, 0)),
            pl.BlockSpec((1, EXPERT_DIM), lambda i: (0, 0)),
        ],
        out_specs=pl.BlockSpec((_BLK, EXPERT_DIM), lambda i: (i, 0)),
        out_shape=jax.ShapeDtypeStruct((BATCH, EXPERT_DIM), jnp.float32),
    )(pairs, half2d, W, b2d)


@jax.jit
def kernel(expert_id, table, W, b):
    ids = expert_id.astype(jnp.int32)
    pairs = _tc_transpose(table.T)
    prow = ((ids >> 13) << 12) | (ids & (_HALF - 1))
    half = (ids >> 12) & 1
    idx = prow.reshape(BATCH // CHUNK, CHUNK)
    xpair = _sc_gather_fn()(pairs, idx)
    return _tc_linear(xpair, half.reshape(BATCH, 1), W, b.reshape(1, EXPERT_DIM))


# speedup vs baseline: 2.6696x; 1.2529x over previous
import functools

import jax
import jax.numpy as jnp
from jax import lax
from jax.experimental import pallas as pl
from jax.experimental.pallas import tpu as pltpu
from jax.experimental.pallas import tpu_sc as plsc

EXPERT_NUM = 1000000
EXPERT_DIM = 64
PAIR_DIM = 2 * EXPERT_DIM
N_PAIRS = EXPERT_NUM // 2
BATCH = 16384

NC = 2
NS = 16
NW = NC * NS
CHUNK = 128
ROWS_PER_W = BATCH // NW
N_CHUNK = ROWS_PER_W // CHUNK

_TBL = 8192


_TGRID = (EXPERT_NUM + _TBL - 1) // _TBL
_HALF = _TBL // 2
N_PROWS = _TGRID * _HALF


_NFULL = EXPERT_NUM // _TBL


def _tp_body(inl_ref, inr_ref, o_ref):
    x128 = jnp.concatenate([inl_ref[...], inr_ref[...]], axis=0)
    o_ref[...] = jnp.swapaxes(x128, 0, 1)


def _tail_body(alias_ref, in_ref, o_ref):
    del alias_ref
    o_ref[:, :EXPERT_DIM] = jnp.swapaxes(in_ref[...], 0, 1)


def _tc_transpose(tableT):
    main = pl.pallas_call(
        _tp_body,
        grid=(_NFULL,),
        in_specs=[
            pl.BlockSpec((EXPERT_DIM, _HALF), lambda i: (0, 2 * i)),
            pl.BlockSpec((EXPERT_DIM, _HALF), lambda i: (0, 2 * i + 1)),
        ],
        out_specs=pl.BlockSpec((_HALF, PAIR_DIM), lambda i: (i, 0)),
        out_shape=jax.ShapeDtypeStruct((N_PROWS, PAIR_DIM), jnp.float32),
    )(tableT, tableT)
    return pl.pallas_call(
        _tail_body,
        grid=(1,),
        in_specs=[
            pl.BlockSpec((8, PAIR_DIM), lambda i: (0, 0)),
            pl.BlockSpec((EXPERT_DIM, _HALF), lambda i: (0, 2 * _NFULL)),
        ],
        out_specs=pl.BlockSpec((_HALF, PAIR_DIM), lambda i: (_NFULL, 0)),
        out_shape=jax.ShapeDtypeStruct((N_PROWS, PAIR_DIM), jnp.float32),
        input_output_aliases={0: 0},
    )(main, tableT)


def _gather_body(pairs_hbm, idx_hbm, out_hbm, idx_v, rows_v, sem):
    wid = lax.axis_index("s") * NC + lax.axis_index("c")
    blk = wid * N_CHUNK
    pltpu.sync_copy(idx_hbm.at[pl.ds(blk, N_CHUNK)], idx_v)
    copies = [
        pltpu.async_copy(
            pairs_hbm.at[idx_v.at[j]],
            rows_v.at[pl.ds(j * CHUNK, CHUNK)],
            sem,
        )
        for j in range(N_CHUNK)
    ]
    for c in copies:
        c.wait()
    pltpu.sync_copy(rows_v, out_hbm.at[pl.ds(wid * ROWS_PER_W, ROWS_PER_W)])


@functools.cache
def _sc_gather_fn():
    return pl.kernel(
        _gather_body,
        out_type=jax.ShapeDtypeStruct((BATCH, PAIR_DIM), jnp.float32),
        mesh=plsc.VectorSubcoreMesh(
            core_axis_name="c", subcore_axis_name="s", num_cores=NC, num_subcores=NS
        ),
        scratch_types=[
            pltpu.VMEM((N_CHUNK, CHUNK), jnp.int32),
            pltpu.VMEM((ROWS_PER_W, PAIR_DIM), jnp.float32),
            pltpu.SemaphoreType.DMA,
        ],
    )


def _linear_body(pair_ref, half_ref, w_ref, b_ref, o_ref):
    odd = half_ref[...] == 1
    x = jnp.where(odd, pair_ref[:, EXPERT_DIM:], pair_ref[:, :EXPERT_DIM])
    o_ref[...] = (
        lax.dot_general(
            x,
            w_ref[...],
            (((1,), (1,)), ((), ())),
            preferred_element_type=jnp.float32,
        )
        + b_ref[...]
    )


_BLK = 2048


def _tc_linear(pairs, half2d, W, b2d):
    return pl.pallas_call(
        _linear_body,
        grid=(BATCH // _BLK,),
        in_specs=[
            pl.BlockSpec((_BLK, PAIR_DIM), lambda i: (i, 0)),
            pl.BlockSpec((_BLK, 1), lambda i: (i, 0)),
            pl.BlockSpec((EXPERT_DIM, EXPERT_DIM), lambda i: (0, 0)),
            pl.BlockSpec((1, EXPERT_DIM), lambda i: (0, 0)),
        ],
        out_specs=pl.BlockSpec((_BLK, EXPERT_DIM), lambda i: (i, 0)),
        out_shape=jax.ShapeDtypeStruct((BATCH, EXPERT_DIM), jnp.float32),
    )(pairs, half2d, W, b2d)


@jax.jit
def kernel(expert_id, table, W, b):
    ids = expert_id.astype(jnp.int32)
    pairs = _tc_transpose(table.T)
    prow = ((ids >> 13) << 12) | (ids & (_HALF - 1))
    half = (ids >> 12) & 1
    idx = prow.reshape(BATCH // CHUNK, CHUNK)
    xpair = _sc_gather_fn()(pairs, idx)
    return _tc_linear(xpair, half.reshape(BATCH, 1), W, b.reshape(1, EXPERT_DIM))


# speedup vs baseline: 2.7243x; 1.0205x over previous
import functools

import jax
import jax.numpy as jnp
from jax import lax
from jax.experimental import pallas as pl
from jax.experimental.pallas import tpu as pltpu
from jax.experimental.pallas import tpu_sc as plsc

EXPERT_NUM = 1000000
EXPERT_DIM = 64
PAIR_DIM = 2 * EXPERT_DIM
N_PAIRS = EXPERT_NUM // 2
BATCH = 16384

NC = 2
NS = 16
NW = NC * NS
CHUNK = 128
ROWS_PER_W = BATCH // NW
N_CHUNK = ROWS_PER_W // CHUNK

_TBL = 8192


_TGRID = (EXPERT_NUM + _TBL - 1) // _TBL
_HALF = _TBL // 2
N_PROWS = _TGRID * _HALF


_NFULL = EXPERT_NUM // _TBL


def _tp_body(inl_ref, inr_ref, o_ref):
    x128 = jnp.concatenate([inl_ref[...], inr_ref[...]], axis=0)
    o_ref[...] = jnp.swapaxes(x128, 0, 1)


def _tc_transpose(tableT):
    return pl.pallas_call(
        _tp_body,
        grid=(_TGRID,),
        in_specs=[
            pl.BlockSpec((EXPERT_DIM, _HALF), lambda i: (0, 2 * i)),
            pl.BlockSpec(
                (EXPERT_DIM, _HALF),
                lambda i: (0, jnp.minimum(2 * i + 1, 2 * _NFULL)),
            ),
        ],
        out_specs=pl.BlockSpec((_HALF, PAIR_DIM), lambda i: (i, 0)),
        out_shape=jax.ShapeDtypeStruct((N_PROWS, PAIR_DIM), jnp.float32),
    )(tableT, tableT)


def _gather_body(pairs_hbm, idx_hbm, out_hbm, idx_v, rows_v, sem):
    wid = lax.axis_index("s") * NC + lax.axis_index("c")
    blk = wid * N_CHUNK
    pltpu.sync_copy(idx_hbm.at[pl.ds(blk, N_CHUNK)], idx_v)
    copies = [
        pltpu.async_copy(
            pairs_hbm.at[idx_v.at[j]],
            rows_v.at[pl.ds(j * CHUNK, CHUNK)],
            sem,
        )
        for j in range(N_CHUNK)
    ]
    for c in copies:
        c.wait()
    pltpu.sync_copy(rows_v, out_hbm.at[pl.ds(wid * ROWS_PER_W, ROWS_PER_W)])


@functools.cache
def _sc_gather_fn():
    return pl.kernel(
        _gather_body,
        out_type=jax.ShapeDtypeStruct((BATCH, PAIR_DIM), jnp.float32),
        mesh=plsc.VectorSubcoreMesh(
            core_axis_name="c", subcore_axis_name="s", num_cores=NC, num_subcores=NS
        ),
        scratch_types=[
            pltpu.VMEM((N_CHUNK, CHUNK), jnp.int32),
            pltpu.VMEM((ROWS_PER_W, PAIR_DIM), jnp.float32),
            pltpu.SemaphoreType.DMA,
        ],
    )


def _linear_body(pair_ref, half_ref, w_ref, b_ref, o_ref):
    odd = half_ref[...] == 1
    x = jnp.where(odd, pair_ref[:, EXPERT_DIM:], pair_ref[:, :EXPERT_DIM])
    o_ref[...] = (
        lax.dot_general(
            x,
            w_ref[...],
            (((1,), (1,)), ((), ())),
            preferred_element_type=jnp.float32,
        )
        + b_ref[...]
    )


_BLK = 4096


def _tc_linear(pairs, half2d, W, b2d):
    return pl.pallas_call(
        _linear_body,
        grid=(BATCH // _BLK,),
        in_specs=[
            pl.BlockSpec((_BLK, PAIR_DIM), lambda i: (i, 0)),
            pl.BlockSpec((_BLK, 1), lambda i: (i, 0)),
            pl.BlockSpec((EXPERT_DIM, EXPERT_DIM), lambda i: (0, 0)),
            pl.BlockSpec((1, EXPERT_DIM), lambda i: (0, 0)),
        ],
        out_specs=pl.BlockSpec((_BLK, EXPERT_DIM), lambda i: (i, 0)),
        out_shape=jax.ShapeDtypeStruct((BATCH, EXPERT_DIM), jnp.float32),
    )(pairs, half2d, W, b2d)


@jax.jit
def kernel(expert_id, table, W, b):
    ids = expert_id.astype(jnp.int32)
    pairs = _tc_transpose(table.T)
    prow = ((ids >> 13) << 12) | (ids & (_HALF - 1))
    half = (ids >> 12) & 1
    idx = prow.reshape(BATCH // CHUNK, CHUNK)
    xpair = _sc_gather_fn()(pairs, idx)
    return _tc_linear(xpair, half.reshape(BATCH, 1), W, b.reshape(1, EXPERT_DIM))


# speedup vs baseline: 3.0809x; 1.1309x over previous
import functools

import jax
import jax.numpy as jnp
from jax import lax
from jax.experimental import pallas as pl
from jax.experimental.pallas import tpu as pltpu
from jax.experimental.pallas import tpu_sc as plsc

EXPERT_NUM = 1000000
EXPERT_DIM = 64
PAIR_DIM = 2 * EXPERT_DIM
N_PAIRS = EXPERT_NUM // 2
BATCH = 16384

NC = 2
NS = 16
NW = NC * NS
CHUNK = 128
ROWS_PER_W = BATCH // NW
N_CHUNK = ROWS_PER_W // CHUNK

_TBL = 16384
_TBL_BITS = _TBL.bit_length() - 1


_TGRID = (EXPERT_NUM + _TBL - 1) // _TBL
_HALF = _TBL // 2
N_PROWS = _TGRID * _HALF


_NFULL = EXPERT_NUM // _TBL


def _tp_body(inl_ref, inr_ref, o_ref):
    x128 = jnp.concatenate([inl_ref[...], inr_ref[...]], axis=0)
    o_ref[...] = jnp.swapaxes(x128, 0, 1)


def _tc_transpose(tableT):
    return pl.pallas_call(
        _tp_body,
        grid=(_TGRID,),
        in_specs=[
            pl.BlockSpec((EXPERT_DIM, _HALF), lambda i: (0, 2 * i)),
            pl.BlockSpec(
                (EXPERT_DIM, _HALF),
                lambda i: (0, jnp.minimum(2 * i + 1, 2 * _NFULL)),
            ),
        ],
        out_specs=pl.BlockSpec((_HALF, PAIR_DIM), lambda i: (i, 0)),
        out_shape=jax.ShapeDtypeStruct((N_PROWS, PAIR_DIM), jnp.float32),
    )(tableT, tableT)


def _gather_body(pairs_hbm, idx_hbm, out_hbm, idx_v, rows_v, sem):
    wid = lax.axis_index("s") * NC + lax.axis_index("c")
    blk = wid * N_CHUNK
    pltpu.sync_copy(idx_hbm.at[pl.ds(blk, N_CHUNK)], idx_v)
    copies = [
        pltpu.async_copy(
            pairs_hbm.at[idx_v.at[j]],
            rows_v.at[pl.ds(j * CHUNK, CHUNK)],
            sem,
        )
        for j in range(N_CHUNK)
    ]
    for c in copies:
        c.wait()
    pltpu.sync_copy(rows_v, out_hbm.at[pl.ds(wid * ROWS_PER_W, ROWS_PER_W)])


@functools.cache
def _sc_gather_fn():
    return pl.kernel(
        _gather_body,
        out_type=jax.ShapeDtypeStruct((BATCH, PAIR_DIM), jnp.float32),
        mesh=plsc.VectorSubcoreMesh(
            core_axis_name="c", subcore_axis_name="s", num_cores=NC, num_subcores=NS
        ),
        scratch_types=[
            pltpu.VMEM((N_CHUNK, CHUNK), jnp.int32),
            pltpu.VMEM((ROWS_PER_W, PAIR_DIM), jnp.float32),
            pltpu.SemaphoreType.DMA,
        ],
    )


def _linear_body(pair_ref, half_ref, w_ref, b_ref, o_ref):
    odd = half_ref[...] == 1
    x = jnp.where(odd, pair_ref[:, EXPERT_DIM:], pair_ref[:, :EXPERT_DIM])
    o_ref[...] = (
        lax.dot_general(
            x,
            w_ref[...],
            (((1,), (1,)), ((), ())),
            preferred_element_type=jnp.float32,
        )
        + b_ref[...]
    )


_BLK = 4096


def _tc_linear(pairs, half2d, W, b2d):
    return pl.pallas_call(
        _linear_body,
        grid=(BATCH // _BLK,),
        in_specs=[
            pl.BlockSpec((_BLK, PAIR_DIM), lambda i: (i, 0)),
            pl.BlockSpec((_BLK, 1), lambda i: (i, 0)),
            pl.BlockSpec((EXPERT_DIM, EXPERT_DIM), lambda i: (0, 0)),
            pl.BlockSpec((1, EXPERT_DIM), lambda i: (0, 0)),
        ],
        out_specs=pl.BlockSpec((_BLK, EXPERT_DIM), lambda i: (i, 0)),
        out_shape=jax.ShapeDtypeStruct((BATCH, EXPERT_DIM), jnp.float32),
    )(pairs, half2d, W, b2d)


@jax.jit
def kernel(expert_id, table, W, b):
    ids = expert_id.astype(jnp.int32)
    pairs = _tc_transpose(table.T)
    prow = ((ids >> _TBL_BITS) << (_TBL_BITS - 1)) | (ids & (_HALF - 1))
    half = (ids >> (_TBL_BITS - 1)) & 1
    idx = prow.reshape(BATCH // CHUNK, CHUNK)
    xpair = _sc_gather_fn()(pairs, idx)
    return _tc_linear(xpair, half.reshape(BATCH, 1), W, b.reshape(1, EXPERT_DIM))
